# baseline (speedup 1.0000x reference)
import functools

import jax
import jax.numpy as jnp
from jax import lax
from jax.experimental import pallas as pl
from jax.experimental.pallas import tpu as pltpu
from jax.experimental.pallas import tpu_sc as plsc

BATCH = 16384
EMB = 32
LANES = 16
NUSERS = 1000000

_info = plsc.get_sparse_core_info()
_NC = _info.num_cores
_NS = _info.num_subcores
_NW = _NC * _NS
_BPW = BATCH // _NW

_NTILES = NUSERS // 128
_TAIL0 = _NTILES * 128
_JPW = (_NTILES + _NW - 1) // _NW
_NJ = 8
_CW = _NJ * 128
_NCHUNK = (_JPW + _NJ - 1) // _NJ
_NVR = BATCH // LANES


def _scan_select(idx_ref, lo, hi, sel_u, sel_b):
    def body(r, cnt):
        b0 = pl.multiple_of(r * LANES, LANES)
        uvec = idx_ref[pl.ds(b0, LANES)]
        bvec = lax.iota(jnp.int32, LANES) + b0
        m = (uvec >= lo) & (uvec < hi)
        npick = plsc.all_reduce_population_count(m)[0]
        plsc.store_compressed(sel_u.at[pl.ds(cnt, LANES)], uvec, mask=m)
        plsc.store_compressed(sel_b.at[pl.ds(cnt, LANES)], bvec, mask=m)
        return cnt + npick

    return lax.fori_loop(0, _NVR, body, jnp.int32(0))


def _stream_table(tab, idx_ref, rows_out, j_lo, j_hi,
                  sel_u, sel_b, bufs, tmp, sems):
    nsel = _scan_select(idx_ref, j_lo * 128, j_hi * 128, sel_u, sel_b)

    cmax = (_NTILES - _NJ) * 128

    def col0(k):
        return jnp.minimum((j_lo + k * _NJ) * 128, cmax)

    def fetch(k, which):
        c0 = col0(k)
        for i in range(4):
            pltpu.async_copy(
                tab.at[pl.ds(i * 8, 8), pl.ds(c0, _CW)],
                bufs[which][i], sems[which])

    def wait(k, which):
        c0 = col0(k)
        for i in range(4):
            pltpu.make_async_copy(
                tab.at[pl.ds(i * 8, 8), pl.ds(c0, _CW)],
                bufs[which][i], sems[which]).wait()

    fetch(0, 0)

    def chunk_body(k, carry, which):
        @pl.when(k + 1 < _NCHUNK)
        def _():
            fetch(k + 1, 1 - which)

        wait(k, which)
        c0 = col0(k)
        chi = c0 + _CW

        def filt(r, cnt):
            s0 = pl.multiple_of(r * LANES, LANES)
            uvec = sel_u[pl.ds(s0, LANES)]
            bvec = sel_b[pl.ds(s0, LANES)]
            lane = lax.iota(jnp.int32, LANES)
            m = (uvec >= c0) & (uvec < chi) & (lane + s0 < nsel)
            npick = plsc.all_reduce_population_count(m)[0]
            plsc.store_compressed(tmp["cu"].at[pl.ds(cnt, LANES)], uvec - c0, mask=m)
            plsc.store_compressed(tmp["cb"].at[pl.ds(cnt, LANES)], bvec, mask=m)
            return cnt + npick

        nvr_sel = lax.div(nsel + (LANES - 1), LANES)
        ck = lax.fori_loop(0, nvr_sel, filt, jnp.int32(0))

        def group(g, carry2):
            s0 = pl.multiple_of(g * LANES, LANES)
            lane = lax.iota(jnp.int32, LANES)
            valid = lane + s0 < ck
            cvec = plsc.load_gather(tmp["cu"], [lane + s0])
            cvec = jnp.where(valid, cvec, jnp.full((LANES,), cvec[0]))
            bvec = plsc.load_gather(tmp["cb"], [lane + s0])
            bvec = jnp.where(valid, bvec, jnp.full((LANES,), bvec[0]))
            tmp["bidx"][pl.ds(0, LANES)] = bvec
            for i in range(4):
                for e in range(8):
                    val = plsc.load_gather(bufs[which][i], [
                        jnp.full((LANES,), e, jnp.int32), cvec])
                    plsc.store_scatter(
                        tmp["row"], [lane, jnp.full((LANES,), i * 8 + e,
                                                    jnp.int32)], val)
            pltpu.async_copy(
                tmp["row"],
                rows_out.at[tmp["bidx"]],
                sems[2]).wait()
            return carry2

        ngrp = lax.div(ck + (LANES - 1), LANES)
        lax.fori_loop(0, ngrp, group, 0)
        return carry

    def outer(k, carry):
        even = lax.rem(k, 2) == 0

        @pl.when(even)
        def _():
            chunk_body(k, carry, 0)

        @pl.when(jnp.logical_not(even))
        def _():
            chunk_body(k, carry, 1)

        return carry

    lax.fori_loop(0, _NCHUNK, outer, 0)


@functools.partial(
    pl.kernel,
    out_type=(jax.ShapeDtypeStruct((BATCH, 128), jnp.float32),
              jax.ShapeDtypeStruct((BATCH, 128), jnp.float32)),
    mesh=plsc.VectorSubcoreMesh(core_axis_name="c", subcore_axis_name="s"),
    compiler_params=pltpu.CompilerParams(
        needs_layout_passes=False, use_tc_tiling_on_sc=True),
    scratch_types=[
        pltpu.VMEM((BATCH,), jnp.int32),
        pltpu.VMEM((BATCH,), jnp.int32),
        pltpu.VMEM((BATCH,), jnp.int32),
        pltpu.VMEM((2048,), jnp.int32),
        pltpu.VMEM((2048,), jnp.int32),
        pltpu.VMEM((LANES,), jnp.int32),
        pltpu.VMEM((LANES, 128), jnp.float32),
        pltpu.VMEM((8, _CW), jnp.float32),
        pltpu.VMEM((8, _CW), jnp.float32),
        pltpu.VMEM((8, _CW), jnp.float32),
        pltpu.VMEM((8, _CW), jnp.float32),
        pltpu.VMEM((8, _CW), jnp.float32),
        pltpu.VMEM((8, _CW), jnp.float32),
        pltpu.VMEM((8, _CW), jnp.float32),
        pltpu.VMEM((8, _CW), jnp.float32),
        pltpu.SemaphoreType.DMA,
        pltpu.SemaphoreType.DMA,
        pltpu.SemaphoreType.DMA,
    ],
)
def _k1(u_hbm, v_hbm, uet_hbm, iet_hbm, ug_hbm, vg_hbm,
        idxv, sel_u, sel_b, cu, cb, bidx, row,
        a0, a1, a2, a3, b0, b1, b2, b3,
        sem0, sem1, sem2):
    wid = lax.axis_index("s") * _NC + lax.axis_index("c")
    j_lo = wid * _JPW
    j_hi = jnp.minimum(j_lo + _JPW, _NTILES)

    bufs = ((a0, a1, a2, a3), (b0, b1, b2, b3))
    sems = (sem0, sem1, sem2)
    tmp = {"cu": cu, "cb": cb, "bidx": bidx, "row": row}

    pltpu.sync_copy(u_hbm, idxv)
    _stream_table(uet_hbm, idxv, ug_hbm, j_lo, j_hi,
                  sel_u, sel_b, bufs, tmp, sems)
    pltpu.sync_copy(v_hbm, idxv)
    _stream_table(iet_hbm, idxv, vg_hbm, j_lo, j_hi,
                  sel_u, sel_b, bufs, tmp, sems)


@functools.partial(
    pl.kernel,
    out_type=jax.ShapeDtypeStruct((BATCH,), jnp.float32),
    mesh=plsc.VectorSubcoreMesh(core_axis_name="c", subcore_axis_name="s"),
    compiler_params=pltpu.CompilerParams(
        needs_layout_passes=False, use_tc_tiling_on_sc=False),
    scratch_types=[
        pltpu.VMEM((_BPW,), jnp.int32),
        pltpu.VMEM((_BPW,), jnp.int32),
        pltpu.VMEM((_BPW, EMB), jnp.float32),
        pltpu.VMEM((_BPW, EMB), jnp.float32),
        pltpu.VMEM((_BPW,), jnp.float32),
        pltpu.VMEM((_BPW,), jnp.float32),
        pltpu.VMEM((16, 128), jnp.float32),
        pltpu.VMEM((16, 128), jnp.float32),
        pltpu.VMEM((_BPW,), jnp.float32),
        pltpu.SemaphoreType.DMA,
        pltpu.SemaphoreType.DMA,
        pltpu.SemaphoreType.DMA,
        pltpu.SemaphoreType.DMA,
    ],
)
def _k2(u_hbm, v_hbm, ug_hbm, vg_hbm, ub_hbm, ib_hbm, ut_hbm, it_hbm,
        out_hbm,
        idx_u, idx_v, urows, vrows, ubias, ibias, utail, itail, out_v,
        sem_u, sem_v, sem_ub, sem_ib):
    wid = lax.axis_index("s") * _NC + lax.axis_index("c")
    base = wid * _BPW

    pltpu.sync_copy(u_hbm.at[pl.ds(base, _BPW)], idx_u)
    pltpu.sync_copy(v_hbm.at[pl.ds(base, _BPW)], idx_v)
    pltpu.sync_copy(ut_hbm, utail)
    pltpu.sync_copy(it_hbm, itail)

    cub = pltpu.async_copy(ub_hbm.at[idx_u], ubias, sem_ub)
    cib = pltpu.async_copy(ib_hbm.at[idx_v], ibias, sem_ib)
    cu = pltpu.async_copy(
        ug_hbm.at[pl.ds(base, _BPW), pl.ds(0, EMB)], urows, sem_u)
    cv = pltpu.async_copy(
        vg_hbm.at[pl.ds(base, _BPW), pl.ds(0, EMB)], vrows, sem_v)
    cu.wait()
    cv.wait()
    cub.wait()
    cib.wait()

    lane = lax.iota(jnp.int32, LANES)

    def block(i, carry):
        r0 = pl.multiple_of(i * LANES, LANES)
        rows = lane + r0
        uvec = idx_u[pl.ds(r0, LANES)]
        vvec = idx_v[pl.ds(r0, LANES)]
        utl = uvec >= _TAIL0
        vtl = vvec >= _TAIL0
        uloc = jnp.where(utl, uvec - _TAIL0, 0) * EMB
        vloc = jnp.where(vtl, vvec - _TAIL0, 0) * EMB
        acc = ubias[pl.ds(r0, LANES)] + ibias[pl.ds(r0, LANES)]
        for e in range(EMB):
            col = jnp.full((LANES,), e, jnp.int32)
            ue = plsc.load_gather(urows, [rows, col])
            ve = plsc.load_gather(vrows, [rows, col])
            uf = uloc + e
            vf = vloc + e
            ut = plsc.load_gather(
                utail, [lax.shift_right_logical(uf, 7), uf & 127])
            vt = plsc.load_gather(
                itail, [lax.shift_right_logical(vf, 7), vf & 127])
            ue = jnp.where(utl, ut, ue)
            ve = jnp.where(vtl, vt, ve)
            acc = acc + ue * ve
        out_v[pl.ds(r0, LANES)] = acc
        return carry

    lax.fori_loop(0, _BPW // LANES, block, 0)
    pltpu.sync_copy(out_v, out_hbm.at[pl.ds(base, _BPW)])


def kernel(u, v, user_emb, item_emb, user_bias, item_bias):
    u32 = u.astype(jnp.int32)
    v32 = v.astype(jnp.int32)
    ug, vg = _k1(u32, v32, user_emb.T, item_emb.T)
    utail = user_emb[_TAIL0:].reshape(16, 128)
    itail = item_emb[_TAIL0:].reshape(16, 128)
    return _k2(u32, v32, ug, vg,
               user_bias.reshape(-1), item_bias.reshape(-1), utail, itail)


# speedup vs baseline: 1.0261x; 1.0261x over previous
import functools

import jax
import jax.numpy as jnp
from jax import lax
from jax.experimental import pallas as pl
from jax.experimental.pallas import tpu as pltpu
from jax.experimental.pallas import tpu_sc as plsc

BATCH = 16384
EMB = 32
LANES = 16
NUSERS = 1000000

_info = plsc.get_sparse_core_info()
_NC = _info.num_cores
_NS = _info.num_subcores
_NW = _NC * _NS
_BPW = BATCH // _NW

_NTILES = NUSERS // 128
_TAIL0 = _NTILES * 128
_JPW = (_NTILES + _NW - 1) // _NW
_NJ = 8
_CW = _NJ * 128
_NCHUNK = (_JPW + _NJ - 1) // _NJ
_NVR = BATCH // LANES


def _scan_select(idx_ref, lo, hi, sel_u, sel_b):
    def body(r, cnt):
        b0 = pl.multiple_of(r * LANES, LANES)
        uvec = idx_ref[pl.ds(b0, LANES)]
        bvec = lax.iota(jnp.int32, LANES) + b0
        m = (uvec >= lo) & (uvec < hi)
        npick = plsc.all_reduce_population_count(m)[0]
        plsc.store_compressed(sel_u.at[pl.ds(cnt, LANES)], uvec, mask=m)
        plsc.store_compressed(sel_b.at[pl.ds(cnt, LANES)], bvec, mask=m)
        return cnt + npick

    return lax.fori_loop(0, _NVR, body, jnp.int32(0))


def _stream_table(tab, idx_ref, rows_out, j_lo, j_hi,
                  sel_u, sel_b, bufs, tmp, sems):
    cmax = (_NTILES - _NJ) * 128

    def col0(k):
        return jnp.minimum((j_lo + k * _NJ) * 128, cmax)

    def fetch(k, which):
        c0 = col0(k)
        for i in range(4):
            pltpu.async_copy(
                tab.at[pl.ds(i * 8, 8), pl.ds(c0, _CW)],
                bufs[which][i], sems[which])

    def wait(k, which):
        c0 = col0(k)
        for i in range(4):
            pltpu.make_async_copy(
                tab.at[pl.ds(i * 8, 8), pl.ds(c0, _CW)],
                bufs[which][i], sems[which]).wait()

    fetch(0, 0)
    nsel = _scan_select(idx_ref, j_lo * 128, j_hi * 128, sel_u, sel_b)

    def chunk_body(k, carry, which):
        @pl.when(k + 1 < _NCHUNK)
        def _():
            fetch(k + 1, 1 - which)

        wait(k, which)
        c0 = col0(k)
        chi = c0 + _CW

        def filt(r, cnt):
            s0 = pl.multiple_of(r * LANES, LANES)
            uvec = sel_u[pl.ds(s0, LANES)]
            bvec = sel_b[pl.ds(s0, LANES)]
            lane = lax.iota(jnp.int32, LANES)
            m = (uvec >= c0) & (uvec < chi) & (lane + s0 < nsel)
            npick = plsc.all_reduce_population_count(m)[0]
            plsc.store_compressed(tmp["cu"].at[pl.ds(cnt, LANES)], uvec - c0, mask=m)
            plsc.store_compressed(tmp["cb"].at[pl.ds(cnt, LANES)], bvec, mask=m)
            return cnt + npick

        nvr_sel = lax.div(nsel + (LANES - 1), LANES)
        ck = lax.fori_loop(0, nvr_sel, filt, jnp.int32(0))

        def group(g, carry2):
            s0 = pl.multiple_of(g * LANES, LANES)
            lane = lax.iota(jnp.int32, LANES)
            valid = lane + s0 < ck
            cvec = plsc.load_gather(tmp["cu"], [lane + s0])
            cvec = jnp.where(valid, cvec, jnp.full((LANES,), cvec[0]))
            bvec = plsc.load_gather(tmp["cb"], [lane + s0])
            bvec = jnp.where(valid, bvec, jnp.full((LANES,), bvec[0]))
            tmp["bidx"][pl.ds(0, LANES)] = bvec
            for i in range(4):
                for e in range(8):
                    val = plsc.load_gather(bufs[which][i], [
                        jnp.full((LANES,), e, jnp.int32), cvec])
                    plsc.store_scatter(
                        tmp["row"], [lane, jnp.full((LANES,), i * 8 + e,
                                                    jnp.int32)], val)
            pltpu.async_copy(
                tmp["row"],
                rows_out.at[tmp["bidx"]],
                sems[2]).wait()
            return carry2

        ngrp = lax.div(ck + (LANES - 1), LANES)
        lax.fori_loop(0, ngrp, group, 0)
        return carry

    def outer(k, carry):
        even = lax.rem(k, 2) == 0

        @pl.when(even)
        def _():
            chunk_body(k, carry, 0)

        @pl.when(jnp.logical_not(even))
        def _():
            chunk_body(k, carry, 1)

        return carry

    lax.fori_loop(0, _NCHUNK, outer, 0)


@functools.partial(
    pl.kernel,
    out_type=(jax.ShapeDtypeStruct((BATCH, 128), jnp.float32),
              jax.ShapeDtypeStruct((BATCH, 128), jnp.float32)),
    mesh=plsc.VectorSubcoreMesh(core_axis_name="c", subcore_axis_name="s"),
    compiler_params=pltpu.CompilerParams(
        needs_layout_passes=False, use_tc_tiling_on_sc=True),
    scratch_types=[
        pltpu.VMEM((BATCH,), jnp.int32),
        pltpu.VMEM((BATCH,), jnp.int32),
        pltpu.VMEM((BATCH,), jnp.int32),
        pltpu.VMEM((2048,), jnp.int32),
        pltpu.VMEM((2048,), jnp.int32),
        pltpu.VMEM((LANES,), jnp.int32),
        pltpu.VMEM((LANES, 128), jnp.float32),
        pltpu.VMEM((8, _CW), jnp.float32),
        pltpu.VMEM((8, _CW), jnp.float32),
        pltpu.VMEM((8, _CW), jnp.float32),
        pltpu.VMEM((8, _CW), jnp.float32),
        pltpu.VMEM((8, _CW), jnp.float32),
        pltpu.VMEM((8, _CW), jnp.float32),
        pltpu.VMEM((8, _CW), jnp.float32),
        pltpu.VMEM((8, _CW), jnp.float32),
        pltpu.SemaphoreType.DMA,
        pltpu.SemaphoreType.DMA,
        pltpu.SemaphoreType.DMA,
    ],
)
def _k1(u_hbm, v_hbm, uet_hbm, iet_hbm, ug_hbm, vg_hbm,
        idxv, sel_u, sel_b, cu, cb, bidx, row,
        a0, a1, a2, a3, b0, b1, b2, b3,
        sem0, sem1, sem2):
    wid = lax.axis_index("s") * _NC + lax.axis_index("c")
    j_lo = wid * _JPW
    j_hi = jnp.minimum(j_lo + _JPW, _NTILES)

    bufs = ((a0, a1, a2, a3), (b0, b1, b2, b3))
    sems = (sem0, sem1, sem2)
    tmp = {"cu": cu, "cb": cb, "bidx": bidx, "row": row}

    pltpu.sync_copy(u_hbm, idxv)
    _stream_table(uet_hbm, idxv, ug_hbm, j_lo, j_hi,
                  sel_u, sel_b, bufs, tmp, sems)
    pltpu.sync_copy(v_hbm, idxv)
    _stream_table(iet_hbm, idxv, vg_hbm, j_lo, j_hi,
                  sel_u, sel_b, bufs, tmp, sems)


@functools.partial(
    pl.kernel,
    out_type=jax.ShapeDtypeStruct((BATCH,), jnp.float32),
    mesh=plsc.VectorSubcoreMesh(core_axis_name="c", subcore_axis_name="s"),
    compiler_params=pltpu.CompilerParams(
        needs_layout_passes=False, use_tc_tiling_on_sc=False),
    scratch_types=[
        pltpu.VMEM((_BPW,), jnp.int32),
        pltpu.VMEM((_BPW,), jnp.int32),
        pltpu.VMEM((_BPW, EMB), jnp.float32),
        pltpu.VMEM((_BPW, EMB), jnp.float32),
        pltpu.VMEM((_BPW,), jnp.float32),
        pltpu.VMEM((_BPW,), jnp.float32),
        pltpu.VMEM((16, 128), jnp.float32),
        pltpu.VMEM((16, 128), jnp.float32),
        pltpu.VMEM((_BPW,), jnp.float32),
        pltpu.SemaphoreType.DMA,
        pltpu.SemaphoreType.DMA,
        pltpu.SemaphoreType.DMA,
        pltpu.SemaphoreType.DMA,
    ],
)
def _k2(u_hbm, v_hbm, ug_hbm, vg_hbm, ub_hbm, ib_hbm, ut_hbm, it_hbm,
        out_hbm,
        idx_u, idx_v, urows, vrows, ubias, ibias, utail, itail, out_v,
        sem_u, sem_v, sem_ub, sem_ib):
    wid = lax.axis_index("s") * _NC + lax.axis_index("c")
    base = wid * _BPW

    pltpu.sync_copy(u_hbm.at[pl.ds(base, _BPW)], idx_u)
    pltpu.sync_copy(v_hbm.at[pl.ds(base, _BPW)], idx_v)
    pltpu.sync_copy(ut_hbm, utail)
    pltpu.sync_copy(it_hbm, itail)

    cub = pltpu.async_copy(ub_hbm.at[idx_u], ubias, sem_ub)
    cib = pltpu.async_copy(ib_hbm.at[idx_v], ibias, sem_ib)
    cu = pltpu.async_copy(
        ug_hbm.at[pl.ds(base, _BPW), pl.ds(0, EMB)], urows, sem_u)
    cv = pltpu.async_copy(
        vg_hbm.at[pl.ds(base, _BPW), pl.ds(0, EMB)], vrows, sem_v)
    cu.wait()
    cv.wait()
    cub.wait()
    cib.wait()

    lane = lax.iota(jnp.int32, LANES)

    def block(i, carry):
        r0 = pl.multiple_of(i * LANES, LANES)
        rows = lane + r0
        uvec = idx_u[pl.ds(r0, LANES)]
        vvec = idx_v[pl.ds(r0, LANES)]
        utl = uvec >= _TAIL0
        vtl = vvec >= _TAIL0
        acc = ubias[pl.ds(r0, LANES)] + ibias[pl.ds(r0, LANES)]
        for e in range(EMB):
            col = jnp.full((LANES,), e, jnp.int32)
            ue = plsc.load_gather(urows, [rows, col])
            ve = plsc.load_gather(vrows, [rows, col])
            acc = acc + ue * ve
        out_v[pl.ds(r0, LANES)] = acc

        anytail = plsc.all_reduce_population_count(utl | vtl)[0] > 0

        @pl.when(anytail)
        def _():
            uloc = jnp.where(utl, uvec - _TAIL0, 0) * EMB
            vloc = jnp.where(vtl, vvec - _TAIL0, 0) * EMB
            acc2 = ubias[pl.ds(r0, LANES)] + ibias[pl.ds(r0, LANES)]
            for e in range(EMB):
                col = jnp.full((LANES,), e, jnp.int32)
                ue = plsc.load_gather(urows, [rows, col])
                ve = plsc.load_gather(vrows, [rows, col])
                uf = uloc + e
                vf = vloc + e
                ut = plsc.load_gather(
                    utail, [lax.shift_right_logical(uf, 7), uf & 127])
                vt = plsc.load_gather(
                    itail, [lax.shift_right_logical(vf, 7), vf & 127])
                ue = jnp.where(utl, ut, ue)
                ve = jnp.where(vtl, vt, ve)
                acc2 = acc2 + ue * ve
            out_v[pl.ds(r0, LANES)] = acc2
        return carry

    lax.fori_loop(0, _BPW // LANES, block, 0)
    pltpu.sync_copy(out_v, out_hbm.at[pl.ds(base, _BPW)])


def kernel(u, v, user_emb, item_emb, user_bias, item_bias):
    u32 = u.astype(jnp.int32)
    v32 = v.astype(jnp.int32)
    ug, vg = _k1(u32, v32, user_emb.T, item_emb.T)
    utail = user_emb[_TAIL0:].reshape(16, 128)
    itail = item_emb[_TAIL0:].reshape(16, 128)
    return _k2(u32, v32, ug, vg,
               user_bias.reshape(-1), item_bias.reshape(-1), utail, itail)


# speedup vs baseline: 1.0277x; 1.0016x over previous
import functools

import jax
import jax.numpy as jnp
from jax import lax
from jax.experimental import pallas as pl
from jax.experimental.pallas import tpu as pltpu
from jax.experimental.pallas import tpu_sc as plsc

BATCH = 16384
EMB = 32
LANES = 16
NUSERS = 1000000

_info = plsc.get_sparse_core_info()
_NC = _info.num_cores
_NS = _info.num_subcores
_NW = _NC * _NS
_BPW = BATCH // _NW

_NTILES = NUSERS // 128
_TAIL0 = _NTILES * 128
_JPW = (_NTILES + _NW - 1) // _NW
_NJ = 8
_CW = _NJ * 128
_NCHUNK = (_JPW + _NJ - 1) // _NJ
_NVR = BATCH // LANES


def _scan_select(idx_ref, lo, hi, sel_u, sel_b):
    def body(r, cnt):
        b0 = pl.multiple_of(r * LANES, LANES)
        uvec = idx_ref[pl.ds(b0, LANES)]
        bvec = lax.iota(jnp.int32, LANES) + b0
        m = (uvec >= lo) & (uvec < hi)
        npick = plsc.all_reduce_population_count(m)[0]
        plsc.store_compressed(sel_u.at[pl.ds(cnt, LANES)], uvec, mask=m)
        plsc.store_compressed(sel_b.at[pl.ds(cnt, LANES)], bvec, mask=m)
        return cnt + npick

    return lax.fori_loop(0, _NVR, body, jnp.int32(0))


def _stream_table(tab, idx_ref, rows_out, j_lo, j_hi,
                  sel_u, sel_b, bufs, tmp, sems):
    cmax = (_NTILES - _NJ) * 128

    def col0(k):
        return jnp.minimum((j_lo + k * _NJ) * 128, cmax)

    def fetch(k, which):
        c0 = col0(k)
        for i in range(4):
            pltpu.async_copy(
                tab.at[pl.ds(i * 8, 8), pl.ds(c0, _CW)],
                bufs[which][i], sems[which])

    def wait(k, which):
        c0 = col0(k)
        for i in range(4):
            pltpu.make_async_copy(
                tab.at[pl.ds(i * 8, 8), pl.ds(c0, _CW)],
                bufs[which][i], sems[which]).wait()

    fetch(0, 0)
    nsel = _scan_select(idx_ref, j_lo * 128, j_hi * 128, sel_u, sel_b)

    def chunk_body(k, carry, which):
        @pl.when(k + 1 < _NCHUNK)
        def _():
            fetch(k + 1, 1 - which)

        wait(k, which)
        c0 = col0(k)
        chi = c0 + _CW

        def filt(r, cnt):
            s0 = pl.multiple_of(r * LANES, LANES)
            uvec = sel_u[pl.ds(s0, LANES)]
            bvec = sel_b[pl.ds(s0, LANES)]
            lane = lax.iota(jnp.int32, LANES)
            m = (uvec >= c0) & (uvec < chi) & (lane + s0 < nsel)
            npick = plsc.all_reduce_population_count(m)[0]
            plsc.store_compressed(tmp["cu"].at[pl.ds(cnt, LANES)], uvec - c0, mask=m)
            plsc.store_compressed(tmp["cb"].at[pl.ds(cnt, LANES)], bvec, mask=m)
            return cnt + npick

        nvr_sel = lax.div(nsel + (LANES - 1), LANES)
        ck = lax.fori_loop(0, nvr_sel, filt, jnp.int32(0))

        def group(g, carry2):
            @pl.when(g > 0)
            def _():
                pltpu.make_async_copy(
                    tmp["row"], rows_out.at[tmp["bidx"]], sems[2]).wait()

            s0 = pl.multiple_of(g * LANES, LANES)
            lane = lax.iota(jnp.int32, LANES)
            valid = lane + s0 < ck
            cvec = plsc.load_gather(tmp["cu"], [lane + s0])
            cvec = jnp.where(valid, cvec, jnp.full((LANES,), cvec[0]))
            bvec = plsc.load_gather(tmp["cb"], [lane + s0])
            bvec = jnp.where(valid, bvec, jnp.full((LANES,), bvec[0]))
            tmp["bidx"][pl.ds(0, LANES)] = bvec
            for i in range(4):
                for e in range(8):
                    val = plsc.load_gather(bufs[which][i], [
                        jnp.full((LANES,), e, jnp.int32), cvec])
                    plsc.store_scatter(
                        tmp["row"], [lane, jnp.full((LANES,), i * 8 + e,
                                                    jnp.int32)], val)
            pltpu.async_copy(
                tmp["row"],
                rows_out.at[tmp["bidx"]],
                sems[2])
            return carry2

        ngrp = lax.div(ck + (LANES - 1), LANES)
        lax.fori_loop(0, ngrp, group, 0)

        @pl.when(ngrp > 0)
        def _():
            pltpu.make_async_copy(
                tmp["row"], rows_out.at[tmp["bidx"]], sems[2]).wait()
        return carry

    def outer(k, carry):
        even = lax.rem(k, 2) == 0

        @pl.when(even)
        def _():
            chunk_body(k, carry, 0)

        @pl.when(jnp.logical_not(even))
        def _():
            chunk_body(k, carry, 1)

        return carry

    lax.fori_loop(0, _NCHUNK, outer, 0)


@functools.partial(
    pl.kernel,
    out_type=(jax.ShapeDtypeStruct((BATCH, 128), jnp.float32),
              jax.ShapeDtypeStruct((BATCH, 128), jnp.float32)),
    mesh=plsc.VectorSubcoreMesh(core_axis_name="c", subcore_axis_name="s"),
    compiler_params=pltpu.CompilerParams(
        needs_layout_passes=False, use_tc_tiling_on_sc=True),
    scratch_types=[
        pltpu.VMEM((BATCH,), jnp.int32),
        pltpu.VMEM((BATCH,), jnp.int32),
        pltpu.VMEM((BATCH,), jnp.int32),
        pltpu.VMEM((2048,), jnp.int32),
        pltpu.VMEM((2048,), jnp.int32),
        pltpu.VMEM((LANES,), jnp.int32),
        pltpu.VMEM((LANES, 128), jnp.float32),
        pltpu.VMEM((8, _CW), jnp.float32),
        pltpu.VMEM((8, _CW), jnp.float32),
        pltpu.VMEM((8, _CW), jnp.float32),
        pltpu.VMEM((8, _CW), jnp.float32),
        pltpu.VMEM((8, _CW), jnp.float32),
        pltpu.VMEM((8, _CW), jnp.float32),
        pltpu.VMEM((8, _CW), jnp.float32),
        pltpu.VMEM((8, _CW), jnp.float32),
        pltpu.SemaphoreType.DMA,
        pltpu.SemaphoreType.DMA,
        pltpu.SemaphoreType.DMA,
    ],
)
def _k1(u_hbm, v_hbm, uet_hbm, iet_hbm, ug_hbm, vg_hbm,
        idxv, sel_u, sel_b, cu, cb, bidx, row,
        a0, a1, a2, a3, b0, b1, b2, b3,
        sem0, sem1, sem2):
    wid = lax.axis_index("s") * _NC + lax.axis_index("c")
    j_lo = wid * _JPW
    j_hi = jnp.minimum(j_lo + _JPW, _NTILES)

    bufs = ((a0, a1, a2, a3), (b0, b1, b2, b3))
    sems = (sem0, sem1, sem2)
    tmp = {"cu": cu, "cb": cb, "bidx": bidx, "row": row}

    pltpu.sync_copy(u_hbm, idxv)
    _stream_table(uet_hbm, idxv, ug_hbm, j_lo, j_hi,
                  sel_u, sel_b, bufs, tmp, sems)
    pltpu.sync_copy(v_hbm, idxv)
    _stream_table(iet_hbm, idxv, vg_hbm, j_lo, j_hi,
                  sel_u, sel_b, bufs, tmp, sems)


@functools.partial(
    pl.kernel,
    out_type=jax.ShapeDtypeStruct((BATCH,), jnp.float32),
    mesh=plsc.VectorSubcoreMesh(core_axis_name="c", subcore_axis_name="s"),
    compiler_params=pltpu.CompilerParams(
        needs_layout_passes=False, use_tc_tiling_on_sc=False),
    scratch_types=[
        pltpu.VMEM((_BPW,), jnp.int32),
        pltpu.VMEM((_BPW,), jnp.int32),
        pltpu.VMEM((_BPW, EMB), jnp.float32),
        pltpu.VMEM((_BPW, EMB), jnp.float32),
        pltpu.VMEM((_BPW,), jnp.float32),
        pltpu.VMEM((_BPW,), jnp.float32),
        pltpu.VMEM((16, 128), jnp.float32),
        pltpu.VMEM((16, 128), jnp.float32),
        pltpu.VMEM((_BPW,), jnp.float32),
        pltpu.SemaphoreType.DMA,
        pltpu.SemaphoreType.DMA,
        pltpu.SemaphoreType.DMA,
        pltpu.SemaphoreType.DMA,
    ],
)
def _k2(u_hbm, v_hbm, ug_hbm, vg_hbm, ub_hbm, ib_hbm, ut_hbm, it_hbm,
        out_hbm,
        idx_u, idx_v, urows, vrows, ubias, ibias, utail, itail, out_v,
        sem_u, sem_v, sem_ub, sem_ib):
    wid = lax.axis_index("s") * _NC + lax.axis_index("c")
    base = wid * _BPW

    pltpu.sync_copy(u_hbm.at[pl.ds(base, _BPW)], idx_u)
    pltpu.sync_copy(v_hbm.at[pl.ds(base, _BPW)], idx_v)
    pltpu.sync_copy(ut_hbm, utail)
    pltpu.sync_copy(it_hbm, itail)

    cub = pltpu.async_copy(ub_hbm.at[idx_u], ubias, sem_ub)
    cib = pltpu.async_copy(ib_hbm.at[idx_v], ibias, sem_ib)
    cu = pltpu.async_copy(
        ug_hbm.at[pl.ds(base, _BPW), pl.ds(0, EMB)], urows, sem_u)
    cv = pltpu.async_copy(
        vg_hbm.at[pl.ds(base, _BPW), pl.ds(0, EMB)], vrows, sem_v)
    cu.wait()
    cv.wait()
    cub.wait()
    cib.wait()

    lane = lax.iota(jnp.int32, LANES)

    def block(i, carry):
        r0 = pl.multiple_of(i * LANES, LANES)
        rows = lane + r0
        uvec = idx_u[pl.ds(r0, LANES)]
        vvec = idx_v[pl.ds(r0, LANES)]
        utl = uvec >= _TAIL0
        vtl = vvec >= _TAIL0
        acc = ubias[pl.ds(r0, LANES)] + ibias[pl.ds(r0, LANES)]
        for e in range(EMB):
            col = jnp.full((LANES,), e, jnp.int32)
            ue = plsc.load_gather(urows, [rows, col])
            ve = plsc.load_gather(vrows, [rows, col])
            acc = acc + ue * ve
        out_v[pl.ds(r0, LANES)] = acc

        anytail = plsc.all_reduce_population_count(utl | vtl)[0] > 0

        @pl.when(anytail)
        def _():
            uloc = jnp.where(utl, uvec - _TAIL0, 0) * EMB
            vloc = jnp.where(vtl, vvec - _TAIL0, 0) * EMB
            acc2 = ubias[pl.ds(r0, LANES)] + ibias[pl.ds(r0, LANES)]
            for e in range(EMB):
                col = jnp.full((LANES,), e, jnp.int32)
                ue = plsc.load_gather(urows, [rows, col])
                ve = plsc.load_gather(vrows, [rows, col])
                uf = uloc + e
                vf = vloc + e
                ut = plsc.load_gather(
                    utail, [lax.shift_right_logical(uf, 7), uf & 127])
                vt = plsc.load_gather(
                    itail, [lax.shift_right_logical(vf, 7), vf & 127])
                ue = jnp.where(utl, ut, ue)
                ve = jnp.where(vtl, vt, ve)
                acc2 = acc2 + ue * ve
            out_v[pl.ds(r0, LANES)] = acc2
        return carry

    lax.fori_loop(0, _BPW // LANES, block, 0)
    pltpu.sync_copy(out_v, out_hbm.at[pl.ds(base, _BPW)])


def kernel(u, v, user_emb, item_emb, user_bias, item_bias):
    u32 = u.astype(jnp.int32)
    v32 = v.astype(jnp.int32)
    ug, vg = _k1(u32, v32, user_emb.T, item_emb.T)
    utail = user_emb[_TAIL0:].reshape(16, 128)
    itail = item_emb[_TAIL0:].reshape(16, 128)
    return _k2(u32, v32, ug, vg,
               user_bias.reshape(-1), item_bias.reshape(-1), utail, itail)
